# baseline (speedup 1.0000x reference)
import jax
import jax.numpy as jnp
from jax.experimental import pallas as pl

B, N = 4, 512
RNN, INTER = 128, 256


def _encoder_kernel(f_ref, w1_ref, b1_ref, w2_ref, b2_ref, w3_ref, b3_ref,
                    out_ref):
    mm = lambda a, b: jax.lax.dot_general(
        a, b, (((1,), (0,)), ((), ())), preferred_element_type=jnp.float32)

    w23 = mm(w2_ref[...], w3_ref[...])
    b23 = mm(b2_ref[...][None, :], w3_ref[...])

    for j in range(B):
        f = f_ref[j]

        deg = jnp.sum(f, axis=0) + 1.0
        dinv = jnp.where(deg > 0.0, jax.lax.rsqrt(deg), 0.0)
        dcol = dinv[:, None]

        def s_apply(x):
            y = x * dcol
            z = jax.lax.dot_general(
                f, y, (((0,), (0,)), ((), ())),
                preferred_element_type=jnp.float32)
            return (z + y) * dcol

        h1 = mm(f, w1_ref[...])
        x1 = s_apply(h1) + b1_ref[...][None, :]
        t1 = s_apply(x1)
        h3 = mm(t1, w23) + b23
        out_ref[j] = s_apply(h3) + b3_ref[...][None, :]


def kernel(flows, W1, b1, W2, b2, W3, b3):
    full = lambda shape: pl.BlockSpec(shape, lambda b: (0,) * len(shape))
    return pl.pallas_call(
        _encoder_kernel,
        grid=(1,),
        in_specs=[
            pl.BlockSpec((B, N, N), lambda b: (0, 0, 0)),
            full((N, RNN)),
            full((RNN,)),
            full((RNN, INTER)),
            full((INTER,)),
            full((INTER, RNN)),
            full((RNN,)),
        ],
        out_specs=pl.BlockSpec((B, N, RNN), lambda b: (0, 0, 0)),
        out_shape=jax.ShapeDtypeStruct((B, N, RNN), jnp.float32),
    )(flows, W1, b1, W2, b2, W3, b3)


# speedup vs baseline: 1.0109x; 1.0109x over previous
import jax
import jax.numpy as jnp
from jax.experimental import pallas as pl

B, N = 4, 512
RNN, INTER = 128, 256


def _encoder_kernel(f_ref, w1_ref, b1_ref, w2_ref, b2_ref, w3_ref, b3_ref,
                    out_ref):
    mm = lambda a, b: jax.lax.dot_general(
        a.astype(jnp.bfloat16), b.astype(jnp.bfloat16),
        (((1,), (0,)), ((), ())), preferred_element_type=jnp.float32)

    w23 = mm(w2_ref[...], w3_ref[...])
    b23 = mm(b2_ref[...][None, :], w3_ref[...])

    for j in range(2):
        f = f_ref[j]

        deg = jnp.sum(f, axis=0) + 1.0
        dinv = jnp.where(deg > 0.0, jax.lax.rsqrt(deg), 0.0)
        dcol = dinv[:, None]
        fb = f.astype(jnp.bfloat16)

        def s_apply(x):
            y = x * dcol
            z = jax.lax.dot_general(
                fb, y.astype(jnp.bfloat16), (((0,), (0,)), ((), ())),
                preferred_element_type=jnp.float32)
            return (z + y) * dcol

        h1 = mm(f, w1_ref[...])
        x1 = s_apply(h1) + b1_ref[...][None, :]
        t1 = s_apply(x1)
        h3 = mm(t1, w23) + b23
        out_ref[j] = s_apply(h3) + b3_ref[...][None, :]


def kernel(flows, W1, b1, W2, b2, W3, b3):
    full = lambda shape: pl.BlockSpec(shape, lambda b: (0,) * len(shape))
    return pl.pallas_call(
        _encoder_kernel,
        grid=(B // 2,),
        in_specs=[
            pl.BlockSpec((2, N, N), lambda b: (b, 0, 0)),
            full((N, RNN)),
            full((RNN,)),
            full((RNN, INTER)),
            full((INTER,)),
            full((INTER, RNN)),
            full((RNN,)),
        ],
        out_specs=pl.BlockSpec((2, N, RNN), lambda b: (b, 0, 0)),
        out_shape=jax.ShapeDtypeStruct((B, N, RNN), jnp.float32),
    )(flows, W1, b1, W2, b2, W3, b3)


# speedup vs baseline: 1.0362x; 1.0250x over previous
import jax
import jax.numpy as jnp
from jax.experimental import pallas as pl

B, N = 4, 512
RNN, INTER = 128, 256


def _encoder_kernel(f_ref, w1_ref, b1_ref, w2_ref, b2_ref, w3_ref, b3_ref,
                    out_ref):
    mm = lambda a, b: jax.lax.dot_general(
        a, b, (((1,), (0,)), ((), ())), preferred_element_type=jnp.float32)

    w23 = mm(w2_ref[...], w3_ref[...])
    b23 = mm(b2_ref[...][None, :], w3_ref[...])

    for j in range(2):
        f = f_ref[j]

        deg = jnp.sum(f, axis=0) + 1.0
        dinv = jnp.where(deg > 0.0, jax.lax.rsqrt(deg), 0.0)
        dsq = dinv * dinv
        fn = f * dinv[:, None] * dinv[None, :]

        def s_apply(x):
            z = jax.lax.dot_general(
                fn, x, (((0,), (0,)), ((), ())),
                preferred_element_type=jnp.float32)
            return z + x * dsq[:, None]

        h1 = mm(f, w1_ref[...])
        x1 = s_apply(h1) + b1_ref[...][None, :]
        t1 = s_apply(x1)
        h3 = mm(t1, w23) + b23
        out_ref[j] = s_apply(h3) + b3_ref[...][None, :]


def kernel(flows, W1, b1, W2, b2, W3, b3):
    full = lambda shape: pl.BlockSpec(shape, lambda b: (0,) * len(shape))
    return pl.pallas_call(
        _encoder_kernel,
        grid=(B // 2,),
        in_specs=[
            pl.BlockSpec((2, N, N), lambda b: (b, 0, 0)),
            full((N, RNN)),
            full((RNN,)),
            full((RNN, INTER)),
            full((INTER,)),
            full((INTER, RNN)),
            full((RNN,)),
        ],
        out_specs=pl.BlockSpec((2, N, RNN), lambda b: (b, 0, 0)),
        out_shape=jax.ShapeDtypeStruct((B, N, RNN), jnp.float32),
    )(flows, W1, b1, W2, b2, W3, b3)
